# baseline (speedup 1.0000x reference)
import functools

import jax
import jax.numpy as jnp
from jax import lax
from jax.experimental import pallas as pl
from jax.experimental.pallas import tpu as pltpu
from jax.experimental.pallas import tpu_sc as plsc

_NC = 2
_NS = 16
_L = 16
_NW = _NC * _NS


def _bf16r(x):
    bits = plsc.bitcast(x, jnp.uint32)
    lsb = (bits >> 16) & jnp.uint32(1)
    bits = (bits + lsb + jnp.uint32(0x7FFF)) & jnp.uint32(0xFFFF0000)
    return plsc.bitcast(bits, jnp.float32)


def _chain_body(n_vocab, seq, bt, table, yflat, params, out,
                idx_v, m_v, par_v, res_v, sem):
    w = lax.axis_index("s") * _NC + lax.axis_index("c")
    base = w * bt

    pltpu.sync_copy(yflat.at[pl.ds(base, bt)], idx_v[0])
    pltpu.sync_copy(params, par_v)

    offs = [0, 1, 2 * n_vocab, 2 * n_vocab + 1]
    for i in range(bt // _L):
        sl = pl.ds(i * _L, _L)
        two_v = idx_v[0][sl] * 2
        for cls in (3, 2, 1):
            idx_v[cls][sl] = two_v + offs[cls]
        idx_v[0][sl] = two_v

    copies = [pltpu.async_copy(table.at[idx_v[cls]], m_v[cls], sem)
              for cls in range(4)]
    for cp in copies:
        cp.wait()

    for cls in range(4):
        for i in range(bt // _L):
            sl = pl.ds(i * _L, _L)
            m_v[cls][sl] = _bf16r(jnp.abs(m_v[cls][sl]))

    a0 = jnp.abs(par_v[pl.ds(0, _L)])
    a1 = jnp.abs(par_v[pl.ds(_L, _L)])
    b0 = jnp.abs(par_v[pl.ds(2 * _L, _L)])
    b1 = jnp.abs(par_v[pl.ds(3 * _L, _L)])

    lane = lax.broadcasted_iota(jnp.int32, (_L,), 0)
    row0 = jnp.minimum(lane, bt // seq - 1) * seq

    def step(t, carry):
        s0, s1 = carry
        r = row0 + t
        s0 = _bf16r(s0)
        s1 = _bf16r(s1)
        m00 = plsc.load_gather(m_v[0], [r])
        m01 = plsc.load_gather(m_v[1], [r])
        m10 = plsc.load_gather(m_v[2], [r])
        m11 = plsc.load_gather(m_v[3], [r])
        return (s0 * m00 + s1 * m10, s0 * m01 + s1 * m11)

    s0, s1 = lax.fori_loop(0, seq, step, (a0, a1))
    res_v[...] = _bf16r(s0) * _bf16r(b0) + _bf16r(s1) * _bf16r(b1)
    pltpu.sync_copy(res_v, out.at[w])


def kernel(y, alpha, beta, core):
    batch, seq = y.shape
    n_vocab = core.shape[2]
    bt = (batch * seq) // _NW

    table = core.reshape(4 * n_vocab)
    yflat = y.reshape(batch * seq)
    params = jnp.broadcast_to(
        jnp.concatenate([alpha[0], beta[0]])[:, None], (4, _L)).reshape(4 * _L)

    chain = pl.kernel(
        functools.partial(_chain_body, n_vocab, seq, bt),
        out_type=jax.ShapeDtypeStruct((_NW, _L), jnp.float32),
        mesh=plsc.VectorSubcoreMesh(core_axis_name="c", subcore_axis_name="s",
                                    num_cores=_NC, num_subcores=_NS),
        compiler_params=pltpu.CompilerParams(needs_layout_passes=False),
        scratch_types=[
            [pltpu.VMEM((bt,), jnp.int32) for _ in range(4)],
            [pltpu.VMEM((bt,), jnp.float32) for _ in range(4)],
            pltpu.VMEM((4 * _L,), jnp.float32),
            pltpu.VMEM((_L,), jnp.float32),
            pltpu.SemaphoreType.DMA,
        ],
    )
    out = chain(table, yflat, params)
    per_batch = bt // seq
    return out[:, :per_batch].reshape(batch)


# speedup vs baseline: 52.9749x; 52.9749x over previous
import functools

import jax
import jax.numpy as jnp
from jax import lax
from jax.experimental import pallas as pl
from jax.experimental.pallas import tpu as pltpu
from jax.experimental.pallas import tpu_sc as plsc

_NC = 2
_NS = 16
_L = 16
_NW = _NC * _NS


def _bf16r(x):
    bits = plsc.bitcast(x, jnp.uint32)
    lsb = (bits >> 16) & jnp.uint32(1)
    bits = (bits + lsb + jnp.uint32(0x7FFF)) & jnp.uint32(0xFFFF0000)
    return plsc.bitcast(bits, jnp.float32)


def _chain_body(n_vocab, seq, bt, table, yflat, params, out,
                idx_v, m_v, par_v, res_v, sem):
    w = lax.axis_index("s") * _NC + lax.axis_index("c")
    base = w * bt

    pltpu.sync_copy(yflat.at[pl.ds(base, bt)], idx_v[0])
    pltpu.sync_copy(params, par_v)

    for i in range(bt // _L):
        sl = pl.ds(i * _L, _L)
        v = idx_v[0][sl]
        for cls in (1, 2, 3):
            idx_v[cls][sl] = v + cls * n_vocab

    copies = [pltpu.async_copy(table.at[idx_v[cls]], m_v[cls], sem)
              for cls in range(4)]
    for cp in copies:
        cp.wait()

    a0 = jnp.abs(par_v[pl.ds(0, _L)])
    a1 = jnp.abs(par_v[pl.ds(_L, _L)])
    b0 = jnp.abs(par_v[pl.ds(2 * _L, _L)])
    b1 = jnp.abs(par_v[pl.ds(3 * _L, _L)])

    lane = lax.broadcasted_iota(jnp.int32, (_L,), 0)
    row0 = jnp.minimum(lane, bt // seq - 1) * seq

    def step(t, carry):
        s0, s1 = carry
        r = row0 + t
        s0 = _bf16r(s0)
        s1 = _bf16r(s1)
        m00 = _bf16r(jnp.abs(plsc.load_gather(m_v[0], [r])))
        m01 = _bf16r(jnp.abs(plsc.load_gather(m_v[1], [r])))
        m10 = _bf16r(jnp.abs(plsc.load_gather(m_v[2], [r])))
        m11 = _bf16r(jnp.abs(plsc.load_gather(m_v[3], [r])))
        return (s0 * m00 + s1 * m10, s0 * m01 + s1 * m11)

    s0, s1 = lax.fori_loop(0, seq, step, (a0, a1))
    res_v[...] = _bf16r(s0) * _bf16r(b0) + _bf16r(s1) * _bf16r(b1)
    pltpu.sync_copy(res_v, out.at[w])


def kernel(y, alpha, beta, core):
    batch, seq = y.shape
    n_vocab = core.shape[2]
    bt = (batch * seq) // _NW

    table = jnp.transpose(core, (0, 1, 3, 2)).reshape(4 * n_vocab)
    yflat = y.reshape(batch * seq)
    params = jnp.broadcast_to(
        jnp.concatenate([alpha[0], beta[0]])[:, None], (4, _L)).reshape(4 * _L)

    chain = pl.kernel(
        functools.partial(_chain_body, n_vocab, seq, bt),
        out_type=jax.ShapeDtypeStruct((_NW, _L), jnp.float32),
        mesh=plsc.VectorSubcoreMesh(core_axis_name="c", subcore_axis_name="s",
                                    num_cores=_NC, num_subcores=_NS),
        compiler_params=pltpu.CompilerParams(needs_layout_passes=False),
        scratch_types=[
            [pltpu.VMEM((bt,), jnp.int32) for _ in range(4)],
            [pltpu.VMEM((bt,), jnp.float32) for _ in range(4)],
            pltpu.VMEM((4 * _L,), jnp.float32),
            pltpu.VMEM((_L,), jnp.float32),
            pltpu.SemaphoreType.DMA,
        ],
    )
    out = chain(table, yflat, params)
    per_batch = bt // seq
    return out[:, :per_batch].reshape(batch)


# speedup vs baseline: 61.6517x; 1.1638x over previous
import functools

import jax
import jax.numpy as jnp
from jax import lax
from jax.experimental import pallas as pl
from jax.experimental.pallas import tpu as pltpu
from jax.experimental.pallas import tpu_sc as plsc

_NC = 2
_NS = 16
_L = 16
_NW = _NC * _NS


def _bf16r(x):
    bits = plsc.bitcast(x, jnp.uint32)
    lsb = (bits >> 16) & jnp.uint32(1)
    bits = (bits + lsb + jnp.uint32(0x7FFF)) & jnp.uint32(0xFFFF0000)
    return plsc.bitcast(bits, jnp.float32)


_CH = 31232
_NCHUNK = 4
_SPAN = _CH * _NCHUNK


def _detile_body(n_vocab, src, tail, dst, buf, sem):
    w = lax.axis_index("s") * _NC + lax.axis_index("c")
    cls = w // 8
    k = w % 8
    v0 = k * _SPAN
    base = cls * n_vocab + v0
    plane = src.at[0, cls // 2, cls % 2]
    for j in range(_NCHUNK):
        pltpu.sync_copy(plane.at[pl.ds(v0 + j * _CH, _CH)], buf)
        pltpu.sync_copy(buf, dst.at[pl.ds(base + j * _CH, _CH)])

    t0 = 8 * _SPAN
    full = (n_vocab - t0) // 128 * 128
    rag = n_vocab - t0 - full

    @pl.when(k == 7)
    def _():
        pltpu.sync_copy(plane.at[pl.ds(t0, full)], buf.at[pl.ds(0, full)])
        pltpu.sync_copy(buf.at[pl.ds(0, full)],
                        dst.at[pl.ds(cls * n_vocab + t0, full)])
        pltpu.sync_copy(tail.at[pl.ds(cls * rag, rag)],
                        buf.at[pl.ds(0, rag)])
        pltpu.sync_copy(buf.at[pl.ds(0, rag)],
                        dst.at[pl.ds(cls * n_vocab + t0 + full, rag)])

    del sem


def _chain_body(n_vocab, seq, bt, table, yflat, params, out,
                idx_v, m_v, par_v, res_v, sem):
    w = lax.axis_index("s") * _NC + lax.axis_index("c")
    base = w * bt

    pltpu.sync_copy(yflat.at[pl.ds(base, bt)], idx_v[0])
    pltpu.sync_copy(params, par_v)

    for i in range(bt // _L):
        sl = pl.ds(i * _L, _L)
        v = idx_v[0][sl]
        for cls in (1, 2, 3):
            idx_v[cls][sl] = v + cls * n_vocab

    copies = [pltpu.async_copy(table.at[idx_v[cls]], m_v[cls], sem)
              for cls in range(4)]
    for cp in copies:
        cp.wait()

    a0 = jnp.abs(par_v[pl.ds(0, _L)])
    a1 = jnp.abs(par_v[pl.ds(_L, _L)])
    b0 = jnp.abs(par_v[pl.ds(2 * _L, _L)])
    b1 = jnp.abs(par_v[pl.ds(3 * _L, _L)])

    lane = lax.broadcasted_iota(jnp.int32, (_L,), 0)
    row0 = jnp.minimum(lane, bt // seq - 1) * seq

    def step(t, carry):
        s0, s1 = carry
        r = row0 + t
        s0 = _bf16r(s0)
        s1 = _bf16r(s1)
        m00 = _bf16r(jnp.abs(plsc.load_gather(m_v[0], [r])))
        m01 = _bf16r(jnp.abs(plsc.load_gather(m_v[1], [r])))
        m10 = _bf16r(jnp.abs(plsc.load_gather(m_v[2], [r])))
        m11 = _bf16r(jnp.abs(plsc.load_gather(m_v[3], [r])))
        return (s0 * m00 + s1 * m10, s0 * m01 + s1 * m11)

    s0, s1 = lax.fori_loop(0, seq, step, (a0, a1))
    res_v[...] = _bf16r(s0) * _bf16r(b0) + _bf16r(s1) * _bf16r(b1)
    pltpu.sync_copy(res_v, out.at[w])


def kernel(y, alpha, beta, core):
    batch, seq = y.shape
    n_vocab = core.shape[2]
    bt = (batch * seq) // _NW

    core_t = jnp.transpose(core, (0, 1, 3, 2))
    rag0 = (8 * _SPAN) + (n_vocab - 8 * _SPAN) // 128 * 128
    tail_flat = core_t[:, :, :, rag0:].reshape(4 * (n_vocab - rag0))
    detile = pl.kernel(
        functools.partial(_detile_body, n_vocab),
        out_type=jax.ShapeDtypeStruct((4 * n_vocab,), jnp.float32),
        mesh=plsc.VectorSubcoreMesh(core_axis_name="c", subcore_axis_name="s",
                                    num_cores=_NC, num_subcores=_NS),
        compiler_params=pltpu.CompilerParams(needs_layout_passes=False),
        scratch_types=[
            pltpu.VMEM((_CH,), jnp.float32),
            pltpu.SemaphoreType.DMA,
        ],
    )
    table = detile(core_t, tail_flat)
    yflat = y.reshape(batch * seq)
    params = jnp.broadcast_to(
        jnp.concatenate([alpha[0], beta[0]])[:, None], (4, _L)).reshape(4 * _L)

    chain = pl.kernel(
        functools.partial(_chain_body, n_vocab, seq, bt),
        out_type=jax.ShapeDtypeStruct((_NW, _L), jnp.float32),
        mesh=plsc.VectorSubcoreMesh(core_axis_name="c", subcore_axis_name="s",
                                    num_cores=_NC, num_subcores=_NS),
        compiler_params=pltpu.CompilerParams(needs_layout_passes=False),
        scratch_types=[
            [pltpu.VMEM((bt,), jnp.int32) for _ in range(4)],
            [pltpu.VMEM((bt,), jnp.float32) for _ in range(4)],
            pltpu.VMEM((4 * _L,), jnp.float32),
            pltpu.VMEM((_L,), jnp.float32),
            pltpu.SemaphoreType.DMA,
        ],
    )
    out = chain(table, yflat, params)
    per_batch = bt // seq
    return out[:, :per_batch].reshape(batch)


# speedup vs baseline: 63.3133x; 1.0270x over previous
import functools

import jax
import jax.numpy as jnp
from jax import lax
from jax.experimental import pallas as pl
from jax.experimental.pallas import tpu as pltpu
from jax.experimental.pallas import tpu_sc as plsc

_NC = 2
_NS = 16
_L = 16
_NW = _NC * _NS


def _bf16r(x):
    bits = plsc.bitcast(x, jnp.uint32)
    lsb = (bits >> 16) & jnp.uint32(1)
    bits = (bits + lsb + jnp.uint32(0x7FFF)) & jnp.uint32(0xFFFF0000)
    return plsc.bitcast(bits, jnp.float32)


_CH = 31232
_NCHUNK = 4
_SPAN = _CH * _NCHUNK


def _detile_body(n_vocab, src, tail, dst, bufs, sems, sem):
    w = lax.axis_index("s") * _NC + lax.axis_index("c")
    cls = w // 8
    k = w % 8
    v0 = k * _SPAN
    base = cls * n_vocab + v0
    plane = src.at[0, cls // 2, cls % 2]
    writes = [None, None]
    for j in range(_NCHUNK):
        b = j % 2
        if writes[b] is not None:
            writes[b].wait()
        pltpu.sync_copy(plane.at[pl.ds(v0 + j * _CH, _CH)], bufs[b])
        writes[b] = pltpu.async_copy(bufs[b], dst.at[pl.ds(base + j * _CH, _CH)],
                                     sems[b])
    for cp in writes:
        cp.wait()
    buf = bufs[0]

    t0 = 8 * _SPAN
    full = (n_vocab - t0) // 128 * 128
    rag = n_vocab - t0 - full

    @pl.when(k == 7)
    def _():
        pltpu.sync_copy(plane.at[pl.ds(t0, full)], buf.at[pl.ds(0, full)])
        pltpu.sync_copy(buf.at[pl.ds(0, full)],
                        dst.at[pl.ds(cls * n_vocab + t0, full)])
        pltpu.sync_copy(tail.at[pl.ds(cls * rag, rag)],
                        buf.at[pl.ds(0, rag)])
        pltpu.sync_copy(buf.at[pl.ds(0, rag)],
                        dst.at[pl.ds(cls * n_vocab + t0 + full, rag)])

    del sem


def _chain_body(n_vocab, seq, bt, table, yflat, params, out,
                idx_v, m_v, par_v, res_v, sem):
    w = lax.axis_index("s") * _NC + lax.axis_index("c")
    base = w * bt

    pltpu.sync_copy(yflat.at[pl.ds(base, bt)], idx_v[0])
    pltpu.sync_copy(params, par_v)

    for i in range(bt // _L):
        sl = pl.ds(i * _L, _L)
        v = idx_v[0][sl]
        for cls in (1, 2, 3):
            idx_v[cls][sl] = v + cls * n_vocab

    copies = [pltpu.async_copy(table.at[idx_v[cls]], m_v[cls], sem)
              for cls in range(4)]
    for cp in copies:
        cp.wait()

    a0 = jnp.abs(par_v[pl.ds(0, _L)])
    a1 = jnp.abs(par_v[pl.ds(_L, _L)])
    b0 = jnp.abs(par_v[pl.ds(2 * _L, _L)])
    b1 = jnp.abs(par_v[pl.ds(3 * _L, _L)])

    lane = lax.broadcasted_iota(jnp.int32, (_L,), 0)
    row0 = jnp.minimum(lane, bt // seq - 1) * seq

    def step(t, carry):
        s0, s1 = carry
        r = row0 + t
        s0 = _bf16r(s0)
        s1 = _bf16r(s1)
        m00 = _bf16r(jnp.abs(plsc.load_gather(m_v[0], [r])))
        m01 = _bf16r(jnp.abs(plsc.load_gather(m_v[1], [r])))
        m10 = _bf16r(jnp.abs(plsc.load_gather(m_v[2], [r])))
        m11 = _bf16r(jnp.abs(plsc.load_gather(m_v[3], [r])))
        return (s0 * m00 + s1 * m10, s0 * m01 + s1 * m11)

    s0, s1 = lax.fori_loop(0, seq, step, (a0, a1))
    res_v[...] = _bf16r(s0) * _bf16r(b0) + _bf16r(s1) * _bf16r(b1)
    pltpu.sync_copy(res_v, out.at[w])


def kernel(y, alpha, beta, core):
    batch, seq = y.shape
    n_vocab = core.shape[2]
    bt = (batch * seq) // _NW

    core_t = jnp.transpose(core, (0, 1, 3, 2))
    rag0 = (8 * _SPAN) + (n_vocab - 8 * _SPAN) // 128 * 128
    tail_flat = core_t[:, :, :, rag0:].reshape(4 * (n_vocab - rag0))
    detile = pl.kernel(
        functools.partial(_detile_body, n_vocab),
        out_type=jax.ShapeDtypeStruct((4 * n_vocab,), jnp.float32),
        mesh=plsc.VectorSubcoreMesh(core_axis_name="c", subcore_axis_name="s",
                                    num_cores=_NC, num_subcores=_NS),
        compiler_params=pltpu.CompilerParams(needs_layout_passes=False),
        scratch_types=[
            [pltpu.VMEM((_CH,), jnp.float32) for _ in range(2)],
            [pltpu.SemaphoreType.DMA for _ in range(2)],
            pltpu.SemaphoreType.DMA,
        ],
    )
    table = detile(core_t, tail_flat)
    yflat = y.reshape(batch * seq)
    params = jnp.broadcast_to(
        jnp.concatenate([alpha[0], beta[0]])[:, None], (4, _L)).reshape(4 * _L)

    chain = pl.kernel(
        functools.partial(_chain_body, n_vocab, seq, bt),
        out_type=jax.ShapeDtypeStruct((_NW, _L), jnp.float32),
        mesh=plsc.VectorSubcoreMesh(core_axis_name="c", subcore_axis_name="s",
                                    num_cores=_NC, num_subcores=_NS),
        compiler_params=pltpu.CompilerParams(needs_layout_passes=False),
        scratch_types=[
            [pltpu.VMEM((bt,), jnp.int32) for _ in range(4)],
            [pltpu.VMEM((bt,), jnp.float32) for _ in range(4)],
            pltpu.VMEM((4 * _L,), jnp.float32),
            pltpu.VMEM((_L,), jnp.float32),
            pltpu.SemaphoreType.DMA,
        ],
    )
    out = chain(table, yflat, params)
    per_batch = bt // seq
    return out[:, :per_batch].reshape(batch)


# speedup vs baseline: 82.7830x; 1.3075x over previous
import functools

import jax
import jax.numpy as jnp
from jax import lax
from jax.experimental import pallas as pl
from jax.experimental.pallas import tpu as pltpu
from jax.experimental.pallas import tpu_sc as plsc

_NC = 2
_NS = 16
_L = 16
_NW = _NC * _NS


def _bf16r(x):
    bits = plsc.bitcast(x, jnp.uint32)
    lsb = (bits >> 16) & jnp.uint32(1)
    bits = (bits + lsb + jnp.uint32(0x7FFF)) & jnp.uint32(0xFFFF0000)
    return plsc.bitcast(bits, jnp.float32)


def _body(n_vocab, seq, bt, src, yflat, params, tail, out,
          idx_v, tail_v, par_v, res_v, bufs, gsem):
    w = lax.axis_index("s") * _NC + lax.axis_index("c")
    base = w * bt
    n_tiles = n_vocab // 128
    v_rag = n_tiles * 128

    pltpu.sync_copy(yflat.at[pl.ds(base, bt)], idx_v)
    pltpu.sync_copy(params, par_v)
    pltpu.sync_copy(tail, tail_v)

    planes = [src.at[0, 0], src.at[0, 1]]

    for c in range(bt // _L):
        chunk = jnp.minimum(idx_v[pl.ds(c * _L, _L)] >> 7, n_tiles - 1) * 128
        for j in range(_L):
            off = pl.multiple_of(chunk[j], 128)
            i = c * _L + j
            for r in range(2):
                pltpu.async_copy(planes[r].at[:, pl.ds(off, 128)],
                                 bufs[r].at[pl.ds(2 * i, 2), :], gsem)
    for _ in range(2 * bt):
        pltpu.make_async_copy(planes[0].at[:, pl.ds(0, 128)],
                              bufs[0].at[pl.ds(0, 2), :], gsem).wait()

    a0 = jnp.abs(par_v[pl.ds(0, _L)])
    a1 = jnp.abs(par_v[pl.ds(_L, _L)])
    b0 = jnp.abs(par_v[pl.ds(2 * _L, _L)])
    b1 = jnp.abs(par_v[pl.ds(3 * _L, _L)])

    lane = lax.broadcasted_iota(jnp.int32, (_L,), 0)
    row0 = jnp.minimum(lane, bt // seq - 1) * seq
    n_rag = n_vocab - v_rag

    def pick(tok_rows, vl, r, s):
        return plsc.load_gather(bufs[r], [tok_rows + s, vl])

    def step(t, carry):
        s0, s1 = carry
        tok = row0 + t
        yv = plsc.load_gather(idx_v, [tok])
        vl = yv & 127
        tok_rows = tok * 2
        in_rag = yv >= v_rag
        ri = jnp.maximum(yv - v_rag, 0)
        m = []
        for cls in range(4):
            r, s = cls // 2, cls % 2
            main = pick(tok_rows, vl, r, s)
            ragv = plsc.load_gather(tail_v, [cls * n_rag + ri])
            m.append(_bf16r(jnp.abs(jnp.where(in_rag, ragv, main))))
        m00, m01, m10, m11 = m
        s0 = _bf16r(s0)
        s1 = _bf16r(s1)
        return (s0 * m00 + s1 * m10, s0 * m01 + s1 * m11)

    s0, s1 = lax.fori_loop(0, seq, step, (a0, a1))
    res_v[...] = _bf16r(s0) * _bf16r(b0) + _bf16r(s1) * _bf16r(b1)
    pltpu.sync_copy(res_v, out.at[w])


def kernel(y, alpha, beta, core):
    batch, seq = y.shape
    n_vocab = core.shape[2]
    bt = (batch * seq) // _NW

    core_t = jnp.transpose(core, (0, 1, 3, 2))
    v_rag = n_vocab // 128 * 128
    n_rag = n_vocab - v_rag
    tail_flat = core_t[:, :, :, v_rag:].reshape(4 * n_rag)
    yflat = y.reshape(batch * seq)
    params = jnp.broadcast_to(
        jnp.concatenate([alpha[0], beta[0]])[:, None], (4, _L)).reshape(4 * _L)

    run = pl.kernel(
        functools.partial(_body, n_vocab, seq, bt),
        out_type=jax.ShapeDtypeStruct((_NW, _L), jnp.float32),
        mesh=plsc.VectorSubcoreMesh(core_axis_name="c", subcore_axis_name="s",
                                    num_cores=_NC, num_subcores=_NS),
        compiler_params=pltpu.CompilerParams(needs_layout_passes=False),
        scratch_types=[
            pltpu.VMEM((bt,), jnp.int32),
            pltpu.VMEM((4 * n_rag,), jnp.float32),
            pltpu.VMEM((4 * _L,), jnp.float32),
            pltpu.VMEM((_L,), jnp.float32),
            [pltpu.VMEM((2 * bt, 128), jnp.float32) for _ in range(2)],
            pltpu.SemaphoreType.DMA,
        ],
    )
    out = run(core_t, yflat, params, tail_flat)
    per_batch = bt // seq
    return out[:, :per_batch].reshape(batch)
